# baseline (speedup 1.0000x reference)
import functools

import jax
import jax.numpy as jnp
from jax import lax
from jax.experimental import pallas as pl
from jax.experimental.pallas import tpu as pltpu
from jax.experimental.pallas import tpu_sc as plsc

_SIZE = 2.0
_RES = 256
_D = 16
_TABLE_SIZE = 2 ** 22
_N = 2000000

_P1 = 2654435761 - (1 << 32)
_P2 = 805459861

_NC = 2
_NS = 16
_NW = _NC * _NS
_B = 2048
_GSZ = 128
_NSUB = _B // _GSZ
_NCHUNK = 31
_NP = _NW * _NCHUNK * _B


def _hash16(x, y, z):
    def quant(v):
        v = v * (_RES / _SIZE) + (_RES / 2.0)
        v = jnp.minimum(jnp.maximum(v, 0.0), float(_RES - 1))
        vi = v.astype(jnp.int32)
        vf = vi.astype(jnp.float32)
        return jnp.where(vf > v, vi - 1, vi)
    i0 = quant(x)
    i1 = quant(y)
    i2 = quant(z)
    h = i0 ^ (i1 * jnp.int32(_P1)) ^ (i2 * jnp.int32(_P2))
    return h & jnp.int32(_TABLE_SIZE - 1)


def _sc_body(x_hbm, y_hbm, z_hbm, table_hbm, out_hbm, xb, yb, zb,
             hbuf, rows, sem):
    wid = lax.axis_index("s") * _NC + lax.axis_index("c")

    def chunk_body(c, carry):
        row_base = (wid * _NCHUNK + c) * _B
        pltpu.sync_copy(x_hbm.at[pl.ds(row_base, _B)], xb)
        pltpu.sync_copy(y_hbm.at[pl.ds(row_base, _B)], yb)
        pltpu.sync_copy(z_hbm.at[pl.ds(row_base, _B)], zb)

        def sub_body(j, carry2):
            for k in range(_GSZ // 16):
                off = j * _GSZ + k * 16
                x = xb[pl.ds(off, 16)]
                y = yb[pl.ds(off, 16)]
                z = zb[pl.ds(off, 16)]
                hbuf[j, pl.ds(k * 16, 16)] = _hash16(x, y, z)
            pltpu.make_async_copy(
                table_hbm.at[hbuf.at[j]],
                rows.at[pl.ds(j * _GSZ, _GSZ)],
                sem,
            ).start()
            return carry2

        lax.fori_loop(0, _NSUB, sub_body, 0)
        pltpu.make_async_copy(table_hbm.at[pl.ds(0, _B)], rows, sem).wait()
        pltpu.sync_copy(rows, out_hbm.at[pl.ds(row_base, _B)])
        return carry

    lax.fori_loop(0, _NCHUNK, chunk_body, 0)


def kernel(xyz, table):
    n = xyz.shape[0]
    xyz_t = jnp.pad(xyz.T, ((0, 0), (0, _NP - n)))
    xf, yf, zf = xyz_t[0], xyz_t[1], xyz_t[2]

    mesh = plsc.VectorSubcoreMesh(core_axis_name="c", subcore_axis_name="s")
    run = pl.kernel(
        _sc_body,
        mesh=mesh,
        out_type=jax.ShapeDtypeStruct((_NP, _D), jnp.float32),
        scratch_types=[
            pltpu.VMEM((_B,), jnp.float32),
            pltpu.VMEM((_B,), jnp.float32),
            pltpu.VMEM((_B,), jnp.float32),
            pltpu.VMEM((_NSUB, _GSZ), jnp.int32),
            pltpu.VMEM((_B, _D), jnp.float32),
            pltpu.SemaphoreType.DMA,
        ],
        compiler_params=pltpu.CompilerParams(use_tc_tiling_on_sc=False),
    )
    out = run(xf, yf, zf, table)
    return out[:n]


# speedup vs baseline: 1.4717x; 1.4717x over previous
import functools

import jax
import jax.numpy as jnp
from jax import lax
from jax.experimental import pallas as pl
from jax.experimental.pallas import tpu as pltpu
from jax.experimental.pallas import tpu_sc as plsc

_SIZE = 2.0
_RES = 256
_D = 16
_TABLE_SIZE = 2 ** 22
_N = 2000000

_P1 = 2654435761 - (1 << 32)
_P2 = 805459861

_NC = 2
_NS = 16
_NW = _NC * _NS
_B = 2048
_GSZ = 128
_NSUB = _B // _GSZ
_NCHUNK_FULL = _N // _B
_NTAIL = (_N - _NCHUNK_FULL * _B) // _GSZ


def _hash16(x, y, z):
    def quant(v):
        v = v * (_RES / _SIZE) + (_RES / 2.0)
        v = jnp.minimum(jnp.maximum(v, 0.0), float(_RES - 1))
        vi = v.astype(jnp.int32)
        vf = vi.astype(jnp.float32)
        return jnp.where(vf > v, vi - 1, vi)
    i0 = quant(x)
    i1 = quant(y)
    i2 = quant(z)
    h = i0 ^ (i1 * jnp.int32(_P1)) ^ (i2 * jnp.int32(_P2))
    return h & jnp.int32(_TABLE_SIZE - 1)


def _for_each_worker_chunk(wid, process):
    def chunk_body(i, carry):
        c = wid + i * _NW

        @pl.when(c < _NCHUNK_FULL)
        def _():
            process(c * _B, _B)
        return carry

    lax.fori_loop(0, (_NCHUNK_FULL + _NW - 1) // _NW, chunk_body, 0)

    @pl.when(wid < _NTAIL)
    def _():
        process(_NCHUNK_FULL * _B + wid * _GSZ, _GSZ)


def _gather_body(x_hbm, y_hbm, z_hbm, table_hbm, out_hbm, xb, yb, zb,
                 hbuf, rows, stage, sem_g, sem_o):
    wid = lax.axis_index("s") * _NC + lax.axis_index("c")
    lane = lax.iota(jnp.int32, 16)

    def process(base, npts):
        pltpu.sync_copy(x_hbm.at[pl.ds(base, npts)], xb.at[pl.ds(0, npts)])
        pltpu.sync_copy(y_hbm.at[pl.ds(base, npts)], yb.at[pl.ds(0, npts)])
        pltpu.sync_copy(z_hbm.at[pl.ds(base, npts)], zb.at[pl.ds(0, npts)])

        def sub_body(j, carry2):
            for k in range(_GSZ // 16):
                off = j * _GSZ + k * 16
                x = xb[pl.ds(off, 16)]
                y = yb[pl.ds(off, 16)]
                z = zb[pl.ds(off, 16)]
                hbuf[j, pl.ds(k * 16, 16)] = _hash16(x, y, z)
            pltpu.make_async_copy(
                table_hbm.at[hbuf.at[j]],
                rows.at[pl.ds(j * _GSZ, _GSZ)],
                sem_g,
            ).start()
            return carry2
        lax.fori_loop(0, npts // _GSZ, sub_body, 0)
        pltpu.make_async_copy(
            table_hbm.at[pl.ds(0, npts)], rows.at[pl.ds(0, npts)], sem_g
        ).wait()

        def tr_body(k, carry2):
            ridx = k * 16 + lane
            for d in range(_D):
                v = plsc.load_gather(
                    rows, [ridx, jnp.full((16,), d, jnp.int32)])
                stage[d, pl.ds(k * 16, 16)] = v
            return carry2
        lax.fori_loop(0, npts // 16, tr_body, 0)

        copies = [
            pltpu.make_async_copy(
                stage.at[d, pl.ds(0, npts)],
                out_hbm.at[pl.ds(d * _N + base, npts)],
                sem_o,
            )
            for d in range(_D)
        ]
        for cp in copies:
            cp.start()
        for cp in copies:
            cp.wait()

    _for_each_worker_chunk(wid, process)


def _retile_body(in_hbm, out_hbm, stage, sem_i):
    wid = lax.axis_index("s") * _NC + lax.axis_index("c")

    def process(base, npts):
        copies = [
            pltpu.make_async_copy(
                in_hbm.at[pl.ds(d * _N + base, npts)],
                stage.at[d, pl.ds(0, npts)],
                sem_i,
            )
            for d in range(_D)
        ]
        for cp in copies:
            cp.start()
        for cp in copies:
            cp.wait()
        pltpu.sync_copy(stage.at[:, pl.ds(0, npts)],
                        out_hbm.at[:, pl.ds(base, npts)])

    _for_each_worker_chunk(wid, process)


def kernel(xyz, table):
    n = xyz.shape[0]
    xyz_t = xyz.T
    xf, yf, zf = xyz_t[0], xyz_t[1], xyz_t[2]

    mesh = plsc.VectorSubcoreMesh(core_axis_name="c", subcore_axis_name="s")
    gather = pl.kernel(
        _gather_body,
        mesh=mesh,
        out_type=jax.ShapeDtypeStruct((_D * n,), jnp.float32),
        scratch_types=[
            pltpu.VMEM((_B,), jnp.float32),
            pltpu.VMEM((_B,), jnp.float32),
            pltpu.VMEM((_B,), jnp.float32),
            pltpu.VMEM((_NSUB, _GSZ), jnp.int32),
            pltpu.VMEM((_B, _D), jnp.float32),
            pltpu.VMEM((_D, _B), jnp.float32),
            pltpu.SemaphoreType.DMA,
            pltpu.SemaphoreType.DMA,
        ],
        compiler_params=pltpu.CompilerParams(
            use_tc_tiling_on_sc=False, needs_layout_passes=False),
    )
    planes = gather(xf, yf, zf, table)

    retile = pl.kernel(
        _retile_body,
        mesh=mesh,
        out_type=jax.ShapeDtypeStruct((_D, n), jnp.float32),
        scratch_types=[
            pltpu.VMEM((_D, _B), jnp.float32),
            pltpu.SemaphoreType.DMA,
        ],
        compiler_params=pltpu.CompilerParams(use_tc_tiling_on_sc=True),
    )
    out_t = retile(planes)
    return out_t.T


# speedup vs baseline: 2.1191x; 1.4399x over previous
import functools

import jax
import jax.numpy as jnp
from jax import lax
from jax.experimental import pallas as pl
from jax.experimental.pallas import tpu as pltpu
from jax.experimental.pallas import tpu_sc as plsc

_SIZE = 2.0
_RES = 256
_D = 16
_TABLE_SIZE = 2 ** 22
_N = 2000000

_P1 = 2654435761 - (1 << 32)
_P2 = 805459861

_NC = 2
_NS = 16
_NW = _NC * _NS
_B = 2048
_GSZ = 128
_NSUB = _B // _GSZ
_NCHUNK_FULL = _N // _B
_NTAIL = (_N - _NCHUNK_FULL * _B) // _GSZ


def _hash16(x, y, z):
    def quant(v):
        v = v * (_RES / _SIZE) + (_RES / 2.0)
        v = jnp.minimum(jnp.maximum(v, 0.0), float(_RES - 1))
        vi = v.astype(jnp.int32)
        vf = vi.astype(jnp.float32)
        return jnp.where(vf > v, vi - 1, vi)
    i0 = quant(x)
    i1 = quant(y)
    i2 = quant(z)
    h = i0 ^ (i1 * jnp.int32(_P1)) ^ (i2 * jnp.int32(_P2))
    return h & jnp.int32(_TABLE_SIZE - 1)


_TBLK = 512
_TBLK_PER_W = _TABLE_SIZE // _TBLK // _NW


def _tprep_body(tbl_t_hbm, out_hbm, stage, rows_out):
    wid = lax.axis_index("s") * _NC + lax.axis_index("c")
    ii = lax.iota(jnp.int32, 16)

    def blk_body(i, carry):
        base_e = (wid * _TBLK_PER_W + i) * _TBLK
        pltpu.sync_copy(tbl_t_hbm.at[:, pl.ds(base_e, _TBLK)], stage)

        def g(k, carry2):
            for d in range(_D):
                v = stage[d, pl.ds(k * 16, 16)]
                plsc.store_scatter(rows_out, [k * 256 + ii * 16 + d], v)
            return carry2
        lax.fori_loop(0, _TBLK // 16, g, 0)
        pltpu.sync_copy(rows_out, out_hbm.at[pl.ds(base_e * _D, _TBLK * _D)])
        return carry

    lax.fori_loop(0, _TBLK_PER_W, blk_body, 0)


def _for_each_worker_chunk(wid, process):
    def chunk_body(i, carry):
        c = wid + i * _NW

        @pl.when(c < _NCHUNK_FULL)
        def _():
            process(c * _B, _B)
        return carry

    lax.fori_loop(0, (_NCHUNK_FULL + _NW - 1) // _NW, chunk_body, 0)

    @pl.when(wid < _NTAIL)
    def _():
        process(_NCHUNK_FULL * _B + wid * _GSZ, _GSZ)


def _gather_body(x_hbm, y_hbm, z_hbm, table_hbm, out_hbm, xb, yb, zb,
                 hbuf, rows, stage, sem_g, sem_o):
    wid = lax.axis_index("s") * _NC + lax.axis_index("c")
    lane = lax.iota(jnp.int32, 16)

    def process(base, npts):
        pltpu.sync_copy(x_hbm.at[pl.ds(base, npts)], xb.at[pl.ds(0, npts)])
        pltpu.sync_copy(y_hbm.at[pl.ds(base, npts)], yb.at[pl.ds(0, npts)])
        pltpu.sync_copy(z_hbm.at[pl.ds(base, npts)], zb.at[pl.ds(0, npts)])

        def sub_body(j, carry2):
            for k in range(_GSZ // 16):
                off = j * _GSZ + k * 16
                x = xb[pl.ds(off, 16)]
                y = yb[pl.ds(off, 16)]
                z = zb[pl.ds(off, 16)]
                hbuf[j, pl.ds(k * 16, 16)] = _hash16(x, y, z)
            pltpu.make_async_copy(
                table_hbm.at[hbuf.at[j]],
                rows.at[pl.ds(j * _GSZ, _GSZ)],
                sem_g,
            ).start()
            return carry2
        lax.fori_loop(0, npts // _GSZ, sub_body, 0)
        pltpu.make_async_copy(
            table_hbm.at[pl.ds(0, npts)], rows.at[pl.ds(0, npts)], sem_g
        ).wait()

        def tr_body(k, carry2):
            ridx = k * 16 + lane
            for d in range(_D):
                v = plsc.load_gather(
                    rows, [ridx, jnp.full((16,), d, jnp.int32)])
                stage[d, pl.ds(k * 16, 16)] = v
            return carry2
        lax.fori_loop(0, npts // 16, tr_body, 0)

        copies = [
            pltpu.make_async_copy(
                stage.at[d, pl.ds(0, npts)],
                out_hbm.at[pl.ds(d * _N + base, npts)],
                sem_o,
            )
            for d in range(_D)
        ]
        for cp in copies:
            cp.start()
        for cp in copies:
            cp.wait()

    _for_each_worker_chunk(wid, process)


def _retile_body(in_hbm, out_hbm, stage, sem_i):
    wid = lax.axis_index("s") * _NC + lax.axis_index("c")

    def process(base, npts):
        copies = [
            pltpu.make_async_copy(
                in_hbm.at[pl.ds(d * _N + base, npts)],
                stage.at[d, pl.ds(0, npts)],
                sem_i,
            )
            for d in range(_D)
        ]
        for cp in copies:
            cp.start()
        for cp in copies:
            cp.wait()
        pltpu.sync_copy(stage.at[:, pl.ds(0, npts)],
                        out_hbm.at[:, pl.ds(base, npts)])

    _for_each_worker_chunk(wid, process)


def kernel(xyz, table):
    n = xyz.shape[0]
    xyz_t = xyz.T
    xf, yf, zf = xyz_t[0], xyz_t[1], xyz_t[2]

    mesh = plsc.VectorSubcoreMesh(core_axis_name="c", subcore_axis_name="s")
    tprep = pl.kernel(
        _tprep_body,
        mesh=mesh,
        out_type=jax.ShapeDtypeStruct((_TABLE_SIZE * _D,), jnp.float32),
        scratch_types=[
            pltpu.VMEM((_D, _TBLK), jnp.float32),
            pltpu.VMEM((_TBLK * _D,), jnp.float32),
        ],
        compiler_params=pltpu.CompilerParams(
            use_tc_tiling_on_sc=True, needs_layout_passes=False),
    )
    table_rows = tprep(table.T).reshape(_TABLE_SIZE, _D)

    gather = pl.kernel(
        _gather_body,
        mesh=mesh,
        out_type=jax.ShapeDtypeStruct((_D * n,), jnp.float32),
        scratch_types=[
            pltpu.VMEM((_B,), jnp.float32),
            pltpu.VMEM((_B,), jnp.float32),
            pltpu.VMEM((_B,), jnp.float32),
            pltpu.VMEM((_NSUB, _GSZ), jnp.int32),
            pltpu.VMEM((_B, _D), jnp.float32),
            pltpu.VMEM((_D, _B), jnp.float32),
            pltpu.SemaphoreType.DMA,
            pltpu.SemaphoreType.DMA,
        ],
        compiler_params=pltpu.CompilerParams(
            use_tc_tiling_on_sc=False, needs_layout_passes=False),
    )
    planes = gather(xf, yf, zf, table_rows)

    retile = pl.kernel(
        _retile_body,
        mesh=mesh,
        out_type=jax.ShapeDtypeStruct((_D, n), jnp.float32),
        scratch_types=[
            pltpu.VMEM((_D, _B), jnp.float32),
            pltpu.SemaphoreType.DMA,
        ],
        compiler_params=pltpu.CompilerParams(use_tc_tiling_on_sc=True),
    )
    out_t = retile(planes)
    return out_t.T


# speedup vs baseline: 2.5910x; 1.2227x over previous
import functools

import jax
import jax.numpy as jnp
from jax import lax
from jax.experimental import pallas as pl
from jax.experimental.pallas import tpu as pltpu
from jax.experimental.pallas import tpu_sc as plsc

_SIZE = 2.0
_RES = 256
_D = 16
_TABLE_SIZE = 2 ** 22
_N = 2000000

_P1 = 2654435761 - (1 << 32)
_P2 = 805459861

_NC = 2
_NS = 16
_NW = _NC * _NS
_B = 1024
_GSZ = 128
_NSUB = _B // _GSZ
_NCHUNK_FULL = _N // _B
_NTAIL = (_N - _NCHUNK_FULL * _B) // _GSZ
_NSLOT = (_NCHUNK_FULL + _NW - 1) // _NW

_TBLK = 512
_TBLK_PER_W = _TABLE_SIZE // _TBLK // _NW
_TQ = 4


def _hash16(x, y, z):
    def quant(v):
        v = v * (_RES / _SIZE) + (_RES / 2.0)
        v = jnp.minimum(jnp.maximum(v, 0.0), float(_RES - 1))
        vi = v.astype(jnp.int32)
        vf = vi.astype(jnp.float32)
        return jnp.where(vf > v, vi - 1, vi)
    i0 = quant(x)
    i1 = quant(y)
    i2 = quant(z)
    h = i0 ^ (i1 * jnp.int32(_P1)) ^ (i2 * jnp.int32(_P2))
    return h & jnp.int32(_TABLE_SIZE - 1)


def _tprep_body(tbl_t_hbm, out_hbm,
                st0, st1, st2, st3, ro0, ro1, ro2, ro3,
                si0, si1, si2, si3, so0, so1, so2, so3):
    wid = lax.axis_index("s") * _NC + lax.axis_index("c")
    ii = lax.iota(jnp.int32, 16)
    sts = (st0, st1, st2, st3)
    ros = (ro0, ro1, ro2, ro3)
    sis = (si0, si1, si2, si3)
    sos = (so0, so1, so2, so3)

    def q_body(q, carry):
        bases = [(wid * _TBLK_PER_W + q * _TQ + s) * _TBLK
                 for s in range(_TQ)]
        ins = [pltpu.make_async_copy(
                   tbl_t_hbm.at[:, pl.ds(bases[s], _TBLK)], sts[s], sis[s])
               for s in range(_TQ)]
        outs = [pltpu.make_async_copy(
                    ros[s], out_hbm.at[pl.ds(bases[s] * _D, _TBLK * _D)],
                    sos[s])
                for s in range(_TQ)]
        for cp in ins:
            cp.start()
        for s in range(_TQ):
            ins[s].wait()

            def g(k, carry2):
                for d in range(_D):
                    v = sts[s][d, pl.ds(k * 16, 16)]
                    plsc.store_scatter(ros[s], [k * 256 + ii * 16 + d], v)
                return carry2
            lax.fori_loop(0, _TBLK // 16, g, 0)
            outs[s].start()
        for cp in outs:
            cp.wait()
        return carry

    lax.fori_loop(0, _TBLK_PER_W // _TQ, q_body, 0)


def _gather_chunk(x_hbm, y_hbm, z_hbm, table_hbm, out_hbm, bufs, base):
    xb, yb, zb, hbuf, rows, stage, sem_i, sem_g, sem_p = bufs

    def sub_body(j, carry2):
        for k in range(_GSZ // 16):
            off = j * _GSZ + k * 16
            hbuf[j, pl.ds(k * 16, 16)] = _hash16(
                xb[pl.ds(off, 16)], yb[pl.ds(off, 16)], zb[pl.ds(off, 16)])
        pltpu.make_async_copy(
            table_hbm.at[hbuf.at[j]],
            rows.at[pl.ds(j * _GSZ, _GSZ)],
            sem_g,
        ).start()
        return carry2
    lax.fori_loop(0, _NSUB, sub_body, 0)


def _finish_chunk(table_hbm, out_hbm, bufs, base, lane):
    xb, yb, zb, hbuf, rows, stage, sem_i, sem_g, sem_p = bufs
    pltpu.make_async_copy(
        table_hbm.at[pl.ds(0, _B)], rows, sem_g).wait()

    def tr_body(k, carry2):
        ridx = k * 16 + lane
        for d in range(_D):
            v = plsc.load_gather(rows, [ridx, jnp.full((16,), d, jnp.int32)])
            stage[d, pl.ds(k * 16, 16)] = v
        return carry2
    lax.fori_loop(0, _B // 16, tr_body, 0)

    for d in range(_D):
        pltpu.make_async_copy(
            stage.at[d], out_hbm.at[pl.ds(d * _N + base, _B)], sem_p
        ).start()


def _wait_planes(out_hbm, bufs, base):
    xb, yb, zb, hbuf, rows, stage, sem_i, sem_g, sem_p = bufs
    for d in range(_D):
        pltpu.make_async_copy(
            stage.at[d], out_hbm.at[pl.ds(d * _N + base, _B)], sem_p
        ).wait()


def _stage_xyz(x_hbm, y_hbm, z_hbm, bufs, base, npts):
    xb, yb, zb, hbuf, rows, stage, sem_i, sem_g, sem_p = bufs
    for src, dst in ((x_hbm, xb), (y_hbm, yb), (z_hbm, zb)):
        pltpu.make_async_copy(
            src.at[pl.ds(base, npts)], dst.at[pl.ds(0, npts)], sem_i).start()


def _wait_xyz(x_hbm, y_hbm, z_hbm, bufs, base, npts):
    xb, yb, zb, hbuf, rows, stage, sem_i, sem_g, sem_p = bufs
    for src, dst in ((x_hbm, xb), (y_hbm, yb), (z_hbm, zb)):
        pltpu.make_async_copy(
            src.at[pl.ds(base, npts)], dst.at[pl.ds(0, npts)], sem_i).wait()


def _gather_body(x_hbm, y_hbm, z_hbm, table_hbm, out_hbm,
                 xa, ya, za, ha, ra, sa,
                 xb2, yb2, zb2, hb, rb, sb,
                 ia, ga, pa, ib, gb, pb):
    wid = lax.axis_index("s") * _NC + lax.axis_index("c")
    lane = lax.iota(jnp.int32, 16)
    bufsA = (xa, ya, za, ha, ra, sa, ia, ga, pa)
    bufsB = (xb2, yb2, zb2, hb, rb, sb, ib, gb, pb)

    def pair_body(i, carry):
        c1 = wid + (2 * i) * _NW
        c2 = wid + (2 * i + 1) * _NW
        b1 = c1 * _B
        b2 = c2 * _B

        @pl.when(c1 < _NCHUNK_FULL)
        def _():
            _stage_xyz(x_hbm, y_hbm, z_hbm, bufsA, b1, _B)

        @pl.when(c2 < _NCHUNK_FULL)
        def _():
            _stage_xyz(x_hbm, y_hbm, z_hbm, bufsB, b2, _B)

        @pl.when(c1 < _NCHUNK_FULL)
        def _():
            _wait_xyz(x_hbm, y_hbm, z_hbm, bufsA, b1, _B)
            _gather_chunk(x_hbm, y_hbm, z_hbm, table_hbm, out_hbm, bufsA, b1)

        @pl.when(c2 < _NCHUNK_FULL)
        def _():
            _wait_xyz(x_hbm, y_hbm, z_hbm, bufsB, b2, _B)
            _gather_chunk(x_hbm, y_hbm, z_hbm, table_hbm, out_hbm, bufsB, b2)

        @pl.when(c1 < _NCHUNK_FULL)
        def _():
            _finish_chunk(table_hbm, out_hbm, bufsA, b1, lane)

        @pl.when(c2 < _NCHUNK_FULL)
        def _():
            _finish_chunk(table_hbm, out_hbm, bufsB, b2, lane)

        @pl.when(c1 < _NCHUNK_FULL)
        def _():
            _wait_planes(out_hbm, bufsA, b1)

        @pl.when(c2 < _NCHUNK_FULL)
        def _():
            _wait_planes(out_hbm, bufsB, b2)
        return carry

    lax.fori_loop(0, (_NSLOT + 1) // 2, pair_body, 0)

    @pl.when(wid < _NTAIL)
    def _():
        tb = _NCHUNK_FULL * _B + wid * _GSZ
        xb, yb, zb, hbuf, rows, stage, sem_i, sem_g, sem_p = bufsA
        _stage_xyz(x_hbm, y_hbm, z_hbm, bufsA, tb, _GSZ)
        _wait_xyz(x_hbm, y_hbm, z_hbm, bufsA, tb, _GSZ)
        for k in range(_GSZ // 16):
            hbuf[0, pl.ds(k * 16, 16)] = _hash16(
                xb[pl.ds(k * 16, 16)], yb[pl.ds(k * 16, 16)],
                zb[pl.ds(k * 16, 16)])
        pltpu.make_async_copy(
            table_hbm.at[hbuf.at[0]], rows.at[pl.ds(0, _GSZ)], sem_g).start()
        pltpu.make_async_copy(
            table_hbm.at[pl.ds(0, _GSZ)], rows.at[pl.ds(0, _GSZ)],
            sem_g).wait()

        def tr_body(k, carry2):
            ridx = k * 16 + lane
            for d in range(_D):
                v = plsc.load_gather(
                    rows, [ridx, jnp.full((16,), d, jnp.int32)])
                stage[d, pl.ds(k * 16, 16)] = v
            return carry2
        lax.fori_loop(0, _GSZ // 16, tr_body, 0)
        copies = [
            pltpu.make_async_copy(
                stage.at[d, pl.ds(0, _GSZ)],
                out_hbm.at[pl.ds(d * _N + tb, _GSZ)], sem_p)
            for d in range(_D)
        ]
        for cp in copies:
            cp.start()
        for cp in copies:
            cp.wait()


def _retile_body(in_hbm, out_hbm, stage, sem_i):
    wid = lax.axis_index("s") * _NC + lax.axis_index("c")

    def process(base, npts):
        copies = [
            pltpu.make_async_copy(
                in_hbm.at[pl.ds(d * _N + base, npts)],
                stage.at[d, pl.ds(0, npts)],
                sem_i,
            )
            for d in range(_D)
        ]
        for cp in copies:
            cp.start()
        for cp in copies:
            cp.wait()
        pltpu.sync_copy(stage.at[:, pl.ds(0, npts)],
                        out_hbm.at[:, pl.ds(base, npts)])

    def chunk_body(i, carry):
        c = wid + i * _NW

        @pl.when(c < _NCHUNK_FULL)
        def _():
            process(c * _B, _B)
        return carry

    lax.fori_loop(0, _NSLOT, chunk_body, 0)

    @pl.when(wid < _NTAIL)
    def _():
        process(_NCHUNK_FULL * _B + wid * _GSZ, _GSZ)


def kernel(xyz, table):
    n = xyz.shape[0]
    xyz_t = xyz.T
    xf, yf, zf = xyz_t[0], xyz_t[1], xyz_t[2]

    mesh = plsc.VectorSubcoreMesh(core_axis_name="c", subcore_axis_name="s")
    tprep = pl.kernel(
        _tprep_body,
        mesh=mesh,
        out_type=jax.ShapeDtypeStruct((_TABLE_SIZE * _D,), jnp.float32),
        scratch_types=(
            [pltpu.VMEM((_D, _TBLK), jnp.float32) for _ in range(_TQ)]
            + [pltpu.VMEM((_TBLK * _D,), jnp.float32) for _ in range(_TQ)]
            + [pltpu.SemaphoreType.DMA for _ in range(2 * _TQ)]
        ),
        compiler_params=pltpu.CompilerParams(
            use_tc_tiling_on_sc=True, needs_layout_passes=False),
    )
    table_rows = tprep(table.T).reshape(_TABLE_SIZE, _D)

    chunk_scratch = [
        pltpu.VMEM((_B,), jnp.float32),
        pltpu.VMEM((_B,), jnp.float32),
        pltpu.VMEM((_B,), jnp.float32),
        pltpu.VMEM((_NSUB, _GSZ), jnp.int32),
        pltpu.VMEM((_B, _D), jnp.float32),
        pltpu.VMEM((_D, _B), jnp.float32),
    ]
    gather = pl.kernel(
        _gather_body,
        mesh=mesh,
        out_type=jax.ShapeDtypeStruct((_D * n,), jnp.float32),
        scratch_types=(
            chunk_scratch + chunk_scratch
            + [pltpu.SemaphoreType.DMA for _ in range(6)]
        ),
        compiler_params=pltpu.CompilerParams(
            use_tc_tiling_on_sc=False, needs_layout_passes=False),
    )
    planes = gather(xf, yf, zf, table_rows)

    retile = pl.kernel(
        _retile_body,
        mesh=mesh,
        out_type=jax.ShapeDtypeStruct((_D, n), jnp.float32),
        scratch_types=[
            pltpu.VMEM((_D, _B), jnp.float32),
            pltpu.SemaphoreType.DMA,
        ],
        compiler_params=pltpu.CompilerParams(use_tc_tiling_on_sc=True),
    )
    out_t = retile(planes)
    return out_t.T


# speedup vs baseline: 2.6588x; 1.0262x over previous
import functools

import jax
import jax.numpy as jnp
from jax import lax
from jax.experimental import pallas as pl
from jax.experimental.pallas import tpu as pltpu
from jax.experimental.pallas import tpu_sc as plsc

_SIZE = 2.0
_RES = 256
_D = 16
_TABLE_SIZE = 2 ** 22
_N = 2000000

_P1 = 2654435761 - (1 << 32)
_P2 = 805459861

_NC = 2
_NS = 16
_NW = _NC * _NS
_B = 1024
_GSZ = 128
_NSUB = _B // _GSZ
_NCHUNK_FULL = _N // _B
_NTAIL = (_N - _NCHUNK_FULL * _B) // _GSZ
_NSLOT = (_NCHUNK_FULL + _NW - 1) // _NW

_TBLK = 512
_TBLK_PER_W = _TABLE_SIZE // _TBLK // _NW
_TQ = 4


def _hash16(x, y, z):
    def quant(v):
        v = v * (_RES / _SIZE) + (_RES / 2.0)
        v = jnp.minimum(jnp.maximum(v, 0.0), float(_RES - 1))
        vi = v.astype(jnp.int32)
        vf = vi.astype(jnp.float32)
        return jnp.where(vf > v, vi - 1, vi)
    i0 = quant(x)
    i1 = quant(y)
    i2 = quant(z)
    h = i0 ^ (i1 * jnp.int32(_P1)) ^ (i2 * jnp.int32(_P2))
    return h & jnp.int32(_TABLE_SIZE - 1)


def _tprep_body(tbl_t_hbm, out_hbm,
                st0, st1, st2, st3, ro0, ro1, ro2, ro3,
                si0, si1, si2, si3, so0, so1, so2, so3):
    wid = lax.axis_index("s") * _NC + lax.axis_index("c")
    ii = lax.iota(jnp.int32, 16)
    sts = (st0, st1, st2, st3)
    ros = (ro0, ro1, ro2, ro3)
    sis = (si0, si1, si2, si3)
    sos = (so0, so1, so2, so3)

    def q_body(q, carry):
        bases = [(wid * _TBLK_PER_W + q * _TQ + s) * _TBLK
                 for s in range(_TQ)]
        ins = [pltpu.make_async_copy(
                   tbl_t_hbm.at[:, pl.ds(bases[s], _TBLK)], sts[s], sis[s])
               for s in range(_TQ)]
        outs = [pltpu.make_async_copy(
                    ros[s], out_hbm.at[pl.ds(bases[s] * _D, _TBLK * _D)],
                    sos[s])
                for s in range(_TQ)]
        pre_d = [ii * 16 + d for d in range(_D)]
        for cp in ins:
            cp.start()
        for s in range(_TQ):
            ins[s].wait()

            def g(k, carry2):
                kb = k * 256
                for d in range(_D):
                    v = sts[s][d, pl.ds(k * 16, 16)]
                    plsc.store_scatter(ros[s], [kb + pre_d[d]], v)
                return carry2
            lax.fori_loop(0, _TBLK // 16, g, 0)
            outs[s].start()
        for cp in outs:
            cp.wait()
        return carry

    lax.fori_loop(0, _TBLK_PER_W // _TQ, q_body, 0)


def _gather_chunk(x_hbm, y_hbm, z_hbm, table_hbm, out_hbm, bufs, base):
    xb, yb, zb, hbuf, rows, stage, sem_i, sem_g, sem_p = bufs

    def sub_body(j, carry2):
        for k in range(_GSZ // 16):
            off = j * _GSZ + k * 16
            hbuf[j, pl.ds(k * 16, 16)] = _hash16(
                xb[pl.ds(off, 16)], yb[pl.ds(off, 16)], zb[pl.ds(off, 16)])
        pltpu.make_async_copy(
            table_hbm.at[hbuf.at[j]],
            rows.at[pl.ds(j * _GSZ, _GSZ)],
            sem_g,
        ).start()
        return carry2
    lax.fori_loop(0, _NSUB, sub_body, 0)


def _finish_chunk(table_hbm, out_hbm, bufs, base, lane):
    xb, yb, zb, hbuf, rows, stage, sem_i, sem_g, sem_p = bufs
    pltpu.make_async_copy(
        table_hbm.at[pl.ds(0, _B)], rows, sem_g).wait()

    def tr_body(k, carry2):
        ridx = k * 16 + lane
        for d in range(_D):
            v = plsc.load_gather(rows, [ridx, jnp.full((16,), d, jnp.int32)])
            stage[d, pl.ds(k * 16, 16)] = v
        return carry2
    lax.fori_loop(0, _B // 16, tr_body, 0)

    for d in range(_D):
        pltpu.make_async_copy(
            stage.at[d], out_hbm.at[pl.ds(d * _N + base, _B)], sem_p
        ).start()


def _wait_planes(out_hbm, bufs, base):
    xb, yb, zb, hbuf, rows, stage, sem_i, sem_g, sem_p = bufs
    for d in range(_D):
        pltpu.make_async_copy(
            stage.at[d], out_hbm.at[pl.ds(d * _N + base, _B)], sem_p
        ).wait()


def _stage_xyz(x_hbm, y_hbm, z_hbm, bufs, base, npts):
    xb, yb, zb, hbuf, rows, stage, sem_i, sem_g, sem_p = bufs
    for src, dst in ((x_hbm, xb), (y_hbm, yb), (z_hbm, zb)):
        pltpu.make_async_copy(
            src.at[pl.ds(base, npts)], dst.at[pl.ds(0, npts)], sem_i).start()


def _wait_xyz(x_hbm, y_hbm, z_hbm, bufs, base, npts):
    xb, yb, zb, hbuf, rows, stage, sem_i, sem_g, sem_p = bufs
    for src, dst in ((x_hbm, xb), (y_hbm, yb), (z_hbm, zb)):
        pltpu.make_async_copy(
            src.at[pl.ds(base, npts)], dst.at[pl.ds(0, npts)], sem_i).wait()


def _gather_body(x_hbm, y_hbm, z_hbm, table_hbm, out_hbm,
                 xa, ya, za, ha, ra, sa,
                 xb2, yb2, zb2, hb, rb, sb,
                 ia, ga, pa, ib, gb, pb):
    wid = lax.axis_index("s") * _NC + lax.axis_index("c")
    lane = lax.iota(jnp.int32, 16)
    bufsA = (xa, ya, za, ha, ra, sa, ia, ga, pa)
    bufsB = (xb2, yb2, zb2, hb, rb, sb, ib, gb, pb)

    def pair_body(i, carry):
        c1 = wid + (2 * i) * _NW
        c2 = wid + (2 * i + 1) * _NW
        b1 = c1 * _B
        b2 = c2 * _B

        @pl.when(c1 < _NCHUNK_FULL)
        def _():
            _stage_xyz(x_hbm, y_hbm, z_hbm, bufsA, b1, _B)

        @pl.when(c2 < _NCHUNK_FULL)
        def _():
            _stage_xyz(x_hbm, y_hbm, z_hbm, bufsB, b2, _B)

        @pl.when(c1 < _NCHUNK_FULL)
        def _():
            _wait_xyz(x_hbm, y_hbm, z_hbm, bufsA, b1, _B)
            _gather_chunk(x_hbm, y_hbm, z_hbm, table_hbm, out_hbm, bufsA, b1)

        @pl.when(c2 < _NCHUNK_FULL)
        def _():
            _wait_xyz(x_hbm, y_hbm, z_hbm, bufsB, b2, _B)
            _gather_chunk(x_hbm, y_hbm, z_hbm, table_hbm, out_hbm, bufsB, b2)

        @pl.when(c1 < _NCHUNK_FULL)
        def _():
            _finish_chunk(table_hbm, out_hbm, bufsA, b1, lane)

        @pl.when(c2 < _NCHUNK_FULL)
        def _():
            _finish_chunk(table_hbm, out_hbm, bufsB, b2, lane)

        @pl.when(c1 < _NCHUNK_FULL)
        def _():
            _wait_planes(out_hbm, bufsA, b1)

        @pl.when(c2 < _NCHUNK_FULL)
        def _():
            _wait_planes(out_hbm, bufsB, b2)
        return carry

    lax.fori_loop(0, (_NSLOT + 1) // 2, pair_body, 0)

    @pl.when(wid < _NTAIL)
    def _():
        tb = _NCHUNK_FULL * _B + wid * _GSZ
        xb, yb, zb, hbuf, rows, stage, sem_i, sem_g, sem_p = bufsA
        _stage_xyz(x_hbm, y_hbm, z_hbm, bufsA, tb, _GSZ)
        _wait_xyz(x_hbm, y_hbm, z_hbm, bufsA, tb, _GSZ)
        for k in range(_GSZ // 16):
            hbuf[0, pl.ds(k * 16, 16)] = _hash16(
                xb[pl.ds(k * 16, 16)], yb[pl.ds(k * 16, 16)],
                zb[pl.ds(k * 16, 16)])
        pltpu.make_async_copy(
            table_hbm.at[hbuf.at[0]], rows.at[pl.ds(0, _GSZ)], sem_g).start()
        pltpu.make_async_copy(
            table_hbm.at[pl.ds(0, _GSZ)], rows.at[pl.ds(0, _GSZ)],
            sem_g).wait()

        def tr_body(k, carry2):
            ridx = k * 16 + lane
            for d in range(_D):
                v = plsc.load_gather(
                    rows, [ridx, jnp.full((16,), d, jnp.int32)])
                stage[d, pl.ds(k * 16, 16)] = v
            return carry2
        lax.fori_loop(0, _GSZ // 16, tr_body, 0)
        copies = [
            pltpu.make_async_copy(
                stage.at[d, pl.ds(0, _GSZ)],
                out_hbm.at[pl.ds(d * _N + tb, _GSZ)], sem_p)
            for d in range(_D)
        ]
        for cp in copies:
            cp.start()
        for cp in copies:
            cp.wait()


def _retile_body(in_hbm, out_hbm, stage_a, stage_b, sem_a, sem_b, sem_o):
    wid = lax.axis_index("s") * _NC + lax.axis_index("c")

    def make_in(stage, sem, base, npts):
        return [
            pltpu.make_async_copy(
                in_hbm.at[pl.ds(d * _N + base, npts)],
                stage.at[d, pl.ds(0, npts)],
                sem,
            )
            for d in range(_D)
        ]

    def write_out(stage, base, npts):
        cp = pltpu.make_async_copy(
            stage.at[:, pl.ds(0, npts)],
            out_hbm.at[:, pl.ds(base, npts)], sem_o)
        cp.start()
        return cp

    def pair_body(i, carry):
        c1 = wid + (2 * i) * _NW
        c2 = wid + (2 * i + 1) * _NW

        @pl.when(c1 < _NCHUNK_FULL)
        def _():
            for cp in make_in(stage_a, sem_a, c1 * _B, _B):
                cp.start()

        @pl.when(c2 < _NCHUNK_FULL)
        def _():
            for cp in make_in(stage_b, sem_b, c2 * _B, _B):
                cp.start()

        @pl.when(c1 < _NCHUNK_FULL)
        def _():
            for cp in make_in(stage_a, sem_a, c1 * _B, _B):
                cp.wait()
            write_out(stage_a, c1 * _B, _B).wait()

        @pl.when(c2 < _NCHUNK_FULL)
        def _():
            for cp in make_in(stage_b, sem_b, c2 * _B, _B):
                cp.wait()
            write_out(stage_b, c2 * _B, _B).wait()
        return carry

    lax.fori_loop(0, (_NSLOT + 1) // 2, pair_body, 0)

    @pl.when(wid < _NTAIL)
    def _():
        base = _NCHUNK_FULL * _B + wid * _GSZ
        for cp in make_in(stage_a, sem_a, base, _GSZ):
            cp.start()
        for cp in make_in(stage_a, sem_a, base, _GSZ):
            cp.wait()
        pltpu.sync_copy(stage_a.at[:, pl.ds(0, _GSZ)],
                        out_hbm.at[:, pl.ds(base, _GSZ)])


def kernel(xyz, table):
    n = xyz.shape[0]
    xyz_t = xyz.T
    xf, yf, zf = xyz_t[0], xyz_t[1], xyz_t[2]

    mesh = plsc.VectorSubcoreMesh(core_axis_name="c", subcore_axis_name="s")
    tprep = pl.kernel(
        _tprep_body,
        mesh=mesh,
        out_type=jax.ShapeDtypeStruct((_TABLE_SIZE * _D,), jnp.float32),
        scratch_types=(
            [pltpu.VMEM((_D, _TBLK), jnp.float32) for _ in range(_TQ)]
            + [pltpu.VMEM((_TBLK * _D,), jnp.float32) for _ in range(_TQ)]
            + [pltpu.SemaphoreType.DMA for _ in range(2 * _TQ)]
        ),
        compiler_params=pltpu.CompilerParams(
            use_tc_tiling_on_sc=True, needs_layout_passes=False),
    )
    table_rows = tprep(table.T).reshape(_TABLE_SIZE, _D)

    chunk_scratch = [
        pltpu.VMEM((_B,), jnp.float32),
        pltpu.VMEM((_B,), jnp.float32),
        pltpu.VMEM((_B,), jnp.float32),
        pltpu.VMEM((_NSUB, _GSZ), jnp.int32),
        pltpu.VMEM((_B, _D), jnp.float32),
        pltpu.VMEM((_D, _B), jnp.float32),
    ]
    gather = pl.kernel(
        _gather_body,
        mesh=mesh,
        out_type=jax.ShapeDtypeStruct((_D * n,), jnp.float32),
        scratch_types=(
            chunk_scratch + chunk_scratch
            + [pltpu.SemaphoreType.DMA for _ in range(6)]
        ),
        compiler_params=pltpu.CompilerParams(
            use_tc_tiling_on_sc=False, needs_layout_passes=False),
    )
    planes = gather(xf, yf, zf, table_rows)

    retile = pl.kernel(
        _retile_body,
        mesh=mesh,
        out_type=jax.ShapeDtypeStruct((_D, n), jnp.float32),
        scratch_types=[
            pltpu.VMEM((_D, _B), jnp.float32),
            pltpu.VMEM((_D, _B), jnp.float32),
            pltpu.SemaphoreType.DMA,
            pltpu.SemaphoreType.DMA,
            pltpu.SemaphoreType.DMA,
        ],
        compiler_params=pltpu.CompilerParams(use_tc_tiling_on_sc=True),
    )
    out_t = retile(planes)
    return out_t.T
